# baseline (speedup 1.0000x reference)
import functools
import jax
import jax.numpy as jnp
from jax import lax
from jax.experimental import pallas as pl
from jax.experimental.pallas import tpu as pltpu
from jax.experimental.pallas import tpu_sc as plsc

D = 64
LANES = 128
NW = 32
B = 4096
L = 200
NBUF = 2


def _make_kernel():
    mesh = plsc.VectorSubcoreMesh(core_axis_name="c", subcore_axis_name="s")

    @functools.partial(
        pl.kernel,
        mesh=mesh,
        out_type=jax.ShapeDtypeStruct((L, 8, NW, 8, LANES), jnp.float32),
        scratch_types=[
            pltpu.VMEM((L, LANES), jnp.int32),
            pltpu.VMEM((LANES, D), jnp.float32),
            pltpu.VMEM((LANES, D), jnp.float32),
            pltpu.VMEM((8, 8, LANES), jnp.float32),
            pltpu.VMEM((8, 8, LANES), jnp.float32),
            pltpu.VMEM((2 * (8 + D), 16), jnp.int32),
            pltpu.SemaphoreType.DMA,
            pltpu.SemaphoreType.DMA,
            pltpu.SemaphoreType.DMA,
            pltpu.SemaphoreType.DMA,
        ],
        compiler_params=pltpu.CompilerParams(use_tc_tiling_on_sc=False, needs_layout_passes=False),
    )
    def emb(table_hbm, idx_hbm, out_hbm, idx_v, rv0, rv1, tb0, tb1, cbuf,
            g0, g1, s0, s1):
        cid = lax.axis_index("c")
        sid = lax.axis_index("s")
        wid = sid * 2 + cid
        rows_v = (rv0, rv1)
        tbuf = (tb0, tb1)
        gsem = (g0, g1)
        ssem = (s0, s1)

        pltpu.sync_copy(idx_hbm.at[:, pl.ds(wid * LANES, LANES)], idx_v)

        iota16 = lax.broadcasted_iota(jnp.int32, (16,), 0)
        for h in range(2):
            o = h * (8 + D)
            for j in range(LANES // 16):
                cbuf[o + j, pl.ds(0, 16)] = iota16 + j * 16
            for d in range(D):
                cbuf[o + 8 + d, pl.ds(0, 16)] = jnp.zeros((16,), jnp.int32) + d

        def fire_gather(l, b):
            pltpu.async_copy(table_hbm.at[idx_v.at[l]], rows_v[b], gsem[b])

        def wait_gather(l, b):
            pltpu.make_async_copy(table_hbm.at[idx_v.at[l]], rows_v[b],
                                  gsem[b]).wait()

        def transpose(jj, b):
            r = rows_v[b]
            base = (jj % 2) * (8 + D)
            for k in range(8):
                for di in range(8):
                    d = k * 8 + di
                    dvec = cbuf[base + 8 + d]
                    for j in range(LANES // 16):
                        v = plsc.load_gather(r, [cbuf[base + j], dvec])
                        tbuf[b][k, di, pl.ds(j * 16, 16)] = v

        def fire_stores(l, b):
            for k in range(8):
                pltpu.async_copy(tbuf[b].at[k], out_hbm.at[l, k, wid], ssem[b])

        def wait_stores(l, b):
            for k in range(8):
                pltpu.make_async_copy(tbuf[b].at[k], out_hbm.at[l, k, wid],
                                      ssem[b]).wait()

        fire_gather(0, 0)
        fire_gather(1, 1)

        def body(j, carry):
            for b in range(NBUF):
                l = j * NBUF + b
                wait_gather(l, b)
                transpose(j, b)
                fire_stores(l, b)

                @pl.when(l + NBUF < L)
                def _():
                    fire_gather(l + NBUF, b)

                wait_stores(l, b)
            return carry

        lax.fori_loop(0, L // NBUF, body, 0)

    return emb


def kernel(token_ids, weight):
    out5 = _make_kernel()(weight, token_ids.T.astype(jnp.int32))
    return out5.transpose(2, 4, 0, 1, 3).reshape(B, L, D)


# speedup vs baseline: 2.0730x; 2.0730x over previous
import functools
import jax
import jax.numpy as jnp
from jax import lax
from jax.experimental import pallas as pl
from jax.experimental.pallas import tpu as pltpu
from jax.experimental.pallas import tpu_sc as plsc

D = 64
LANES = 128
CH_VECS = 2
CHUNK = CH_VECS * LANES
NBUF = 2


def _make_kernel(rows, num_workers):
    rows_per_w = rows // num_workers
    nch = rows_per_w // CHUNK
    iv_per_w = rows_per_w // LANES
    assert nch % NBUF == 0

    mesh = plsc.VectorSubcoreMesh(core_axis_name="c", subcore_axis_name="s")

    @functools.partial(
        pl.kernel,
        mesh=mesh,
        out_type=jax.ShapeDtypeStruct((rows, D), jnp.float32),
        scratch_types=[
            pltpu.VMEM((iv_per_w, LANES), jnp.int32),
            pltpu.VMEM((NBUF, CHUNK, D), jnp.float32),
            pltpu.SemaphoreType.DMA,
            pltpu.SemaphoreType.DMA,
            pltpu.SemaphoreType.DMA,
            pltpu.SemaphoreType.DMA,
        ],
        compiler_params=pltpu.CompilerParams(use_tc_tiling_on_sc=False),
    )
    def emb(table_hbm, idx_hbm, out_hbm, idx_v, rows_v, g0, g1, s0, s1):
        cid = lax.axis_index("c")
        sid = lax.axis_index("s")
        wid = sid * 2 + cid
        base = wid * rows_per_w
        ivbase = wid * iv_per_w
        gsem = (g0, g1)
        ssem = (s0, s1)

        pltpu.sync_copy(idx_hbm.at[pl.ds(ivbase, iv_per_w)], idx_v)

        def fire_g(c, b):
            for j in range(CH_VECS):
                pltpu.async_copy(
                    table_hbm.at[idx_v.at[c * CH_VECS + j]],
                    rows_v.at[b, pl.ds(j * LANES, LANES)],
                    gsem[b],
                )

        def drain_g(c, b):
            for j in range(CH_VECS):
                pltpu.make_async_copy(
                    table_hbm.at[idx_v.at[c * CH_VECS + j]],
                    rows_v.at[b, pl.ds(j * LANES, LANES)],
                    gsem[b],
                ).wait()

        def fire_s(c, b):
            pltpu.async_copy(
                rows_v.at[b], out_hbm.at[pl.ds(base + c * CHUNK, CHUNK)], ssem[b]
            )

        def drain_s(c, b):
            pltpu.make_async_copy(
                rows_v.at[b], out_hbm.at[pl.ds(base + c * CHUNK, CHUNK)], ssem[b]
            ).wait()

        fire_g(0, 0)
        fire_g(1, 1)

        def body(g, carry):
            drain_g(g, 0)
            fire_s(g, 0)
            drain_g(g + 1, 1)
            fire_s(g + 1, 1)
            drain_s(g, 0)

            @pl.when(g + 2 < nch)
            def _():
                fire_g(g + 2, 0)

            drain_s(g + 1, 1)

            @pl.when(g + 3 < nch)
            def _():
                fire_g(g + 3, 1)

            return carry

        lax.fori_loop(0, nch // 2, lambda i, c: body(i * 2, c), 0)

    return emb


def kernel(token_ids, weight):
    B, L = token_ids.shape
    rows = B * L
    half = rows // 2
    idx2d = token_ids.T.reshape(rows // LANES, LANES).astype(jnp.int32)
    emb = _make_kernel(half, 32)
    o0 = emb(weight, idx2d[: half // LANES])
    o1 = emb(weight, idx2d[half // LANES:])
    h0 = o0.reshape(L // 2, B, D).transpose(1, 0, 2)
    h1 = o1.reshape(L // 2, B, D).transpose(1, 0, 2)
    return jnp.concatenate([h0, h1], axis=1)


# speedup vs baseline: 2.1674x; 1.0455x over previous
import functools
import jax
import jax.numpy as jnp
from jax import lax
from jax.experimental import pallas as pl
from jax.experimental.pallas import tpu as pltpu
from jax.experimental.pallas import tpu_sc as plsc

D = 64
LANES = 128
CH_VECS = 4
CHUNK = CH_VECS * LANES
NBUF = 2


def _make_kernel(rows, num_workers):
    rows_per_w = rows // num_workers
    nch = rows_per_w // CHUNK
    iv_per_w = rows_per_w // LANES
    assert nch % NBUF == 0

    mesh = plsc.VectorSubcoreMesh(core_axis_name="c", subcore_axis_name="s")

    @functools.partial(
        pl.kernel,
        mesh=mesh,
        out_type=jax.ShapeDtypeStruct((rows, D), jnp.float32),
        scratch_types=[
            pltpu.VMEM((iv_per_w, LANES), jnp.int32),
            pltpu.VMEM((NBUF, CHUNK, D), jnp.float32),
            pltpu.SemaphoreType.DMA,
            pltpu.SemaphoreType.DMA,
            pltpu.SemaphoreType.DMA,
            pltpu.SemaphoreType.DMA,
        ],
        compiler_params=pltpu.CompilerParams(use_tc_tiling_on_sc=False),
    )
    def emb(table_hbm, idx_hbm, out_hbm, idx_v, rows_v, g0, g1, s0, s1):
        cid = lax.axis_index("c")
        sid = lax.axis_index("s")
        wid = sid * 2 + cid
        base = wid * rows_per_w
        ivbase = wid * iv_per_w
        gsem = (g0, g1)
        ssem = (s0, s1)

        pltpu.sync_copy(idx_hbm.at[pl.ds(ivbase, iv_per_w)], idx_v)

        def fire_g(c, b):
            for j in range(CH_VECS):
                pltpu.async_copy(
                    table_hbm.at[idx_v.at[c * CH_VECS + j]],
                    rows_v.at[b, pl.ds(j * LANES, LANES)],
                    gsem[b],
                )

        def drain_g(c, b):
            for j in range(CH_VECS):
                pltpu.make_async_copy(
                    table_hbm.at[idx_v.at[c * CH_VECS + j]],
                    rows_v.at[b, pl.ds(j * LANES, LANES)],
                    gsem[b],
                ).wait()

        def fire_s(c, b):
            pltpu.async_copy(
                rows_v.at[b], out_hbm.at[pl.ds(base + c * CHUNK, CHUNK)], ssem[b]
            )

        def drain_s(c, b):
            pltpu.make_async_copy(
                rows_v.at[b], out_hbm.at[pl.ds(base + c * CHUNK, CHUNK)], ssem[b]
            ).wait()

        fire_g(0, 0)
        fire_g(1, 1)

        def body(g, carry):
            drain_g(g, 0)
            fire_s(g, 0)
            drain_g(g + 1, 1)
            fire_s(g + 1, 1)
            drain_s(g, 0)

            @pl.when(g + 2 < nch)
            def _():
                fire_g(g + 2, 0)

            drain_s(g + 1, 1)

            @pl.when(g + 3 < nch)
            def _():
                fire_g(g + 3, 1)

            return carry

        lax.fori_loop(0, nch // 2, lambda i, c: body(i * 2, c), 0)

    return emb


def kernel(token_ids, weight):
    B, L = token_ids.shape
    rows = B * L
    idx2d = token_ids.reshape(rows // LANES, LANES).astype(jnp.int32)
    emb = _make_kernel(rows, 32)
    out = emb(weight, idx2d)
    return out.reshape(B, L, D)
